# baseline (speedup 1.0000x reference)
import functools

import jax
import jax.numpy as jnp
from jax import lax
from jax.experimental import pallas as pl
from jax.experimental.pallas import tpu as pltpu
from jax.experimental.pallas import tpu_sc as plsc

NC = 2
NS = 16
NW = NC * NS
L = 16

B = 16384
D = 32
BPW = B // NW
CH = 128
NCH = BPW // CH

_mesh = plsc.VectorSubcoreMesh(core_axis_name="c", subcore_axis_name="s")


@functools.partial(
    pl.kernel,
    mesh=_mesh,
    out_type=jax.ShapeDtypeStruct((NW, NCH, CH, D), jnp.float32),
    scratch_types=[
        pltpu.VMEM((NCH, CH), jnp.int32),
        pltpu.VMEM((NCH, CH), jnp.int32),
        pltpu.VMEM((NCH, CH), jnp.int32),
        pltpu.VMEM((NCH, CH, D), jnp.float32),
        pltpu.VMEM((NCH, CH, D), jnp.float32),
        pltpu.VMEM((NCH, CH, D), jnp.float32),
        pltpu.SemaphoreType.DMA,
    ],
    compiler_params=pltpu.CompilerParams(use_tc_tiling_on_sc=False),
)
def _ptranse_sc(e1_hbm, e2_hbm, r_hbm, ent_hbm, rel_hbm, out_hbm,
                i1, i2, ir, r1, r2, rr, sem):
    wid = lax.axis_index("s") * NC + lax.axis_index("c")

    pltpu.sync_copy(e1_hbm.at[wid], i1)
    pltpu.sync_copy(e2_hbm.at[wid], i2)
    pltpu.sync_copy(r_hbm.at[wid], ir)

    copies = []
    for j in range(NCH):
        copies.append(pltpu.async_copy(ent_hbm.at[i1.at[j]], r1.at[j], sem))
        copies.append(pltpu.async_copy(ent_hbm.at[i2.at[j]], r2.at[j], sem))
        copies.append(pltpu.async_copy(rel_hbm.at[ir.at[j]], rr.at[j], sem))
    for c in copies:
        c.wait()

    def row_body(i, _):
        for j in range(NCH):
            for h in range(D // L):
                s = pl.ds(h * L, L)
                r1[j, i, s] = jnp.abs(r1[j, i, s] + rr[j, i, s] - r2[j, i, s])
        return 0

    lax.fori_loop(0, CH, row_body, 0)

    pltpu.sync_copy(r1, out_hbm.at[wid])


def kernel(e1, e2, r, entity_emb, rel_emb):
    e1w = e1.astype(jnp.int32).reshape(NW, NCH, CH)
    e2w = e2.astype(jnp.int32).reshape(NW, NCH, CH)
    rw = r.astype(jnp.int32).reshape(NW, NCH, CH)
    out = _ptranse_sc(e1w, e2w, rw, entity_emb, rel_emb)
    return out.reshape(B, D)
